# baseline (speedup 1.0000x reference)
import functools

import jax
import jax.numpy as jnp
from jax import lax
from jax.experimental import pallas as pl
from jax.experimental.pallas import tpu as pltpu
from jax.experimental.pallas import tpu_sc as plsc

NC = 2
NS = 16
NW = NC * NS
L = 16


def _gather_call(V, D, J, B, NBUF, GD):
    CH = B // NW
    n_ch = J
    WD = NBUF - GD
    assert 1 <= GD < NBUF and n_ch % NBUF == 0
    n_outer = n_ch // NBUF
    assert n_outer >= 2 and CH % L == 0

    scratch = (
        [pltpu.VMEM((CH,), jnp.int32) for _ in range(NBUF)]
        + [pltpu.VMEM((CH, D), jnp.float32) for _ in range(NBUF)]
        + [pltpu.SemaphoreType.DMA for _ in range(2 * NBUF)]
    )

    mesh = plsc.VectorSubcoreMesh(core_axis_name="c", subcore_axis_name="s",
                                  num_cores=NC)

    @functools.partial(
        pl.kernel,
        out_type=jax.ShapeDtypeStruct((J * B, D), jnp.float32),
        mesh=mesh,
        scratch_types=scratch,
        compiler_params=pltpu.CompilerParams(use_tc_tiling_on_sc=False),
    )
    def gather_kernel(table_hbm, idx_hbm, out_hbm, *refs):
        idx_v = refs[0:NBUF]
        rows_v = refs[NBUF:2 * NBUF]
        sem_g = refs[2 * NBUF:3 * NBUF]
        sem_w = refs[3 * NBUF:4 * NBUF]
        wid = lax.axis_index("s") * NC + lax.axis_index("c")
        boff = wid * CH

        def start_gather(c, b):
            pltpu.sync_copy(idx_hbm.at[pl.ds(c * B + boff, CH)], idx_v[b])

            def remap(g, carry):
                v = idx_v[b][pl.ds(g * L, L)]
                p = ((v & -32) | ((v & 7) << 2) | ((v >> 3) & 3))
                idx_v[b][pl.ds(g * L, L)] = p
                return carry

            lax.fori_loop(0, CH // L, remap, 0)
            pltpu.async_copy(table_hbm.at[idx_v[b]], rows_v[b], sem_g[b])

        def wait_gather(b):
            pltpu.make_async_copy(table_hbm.at[idx_v[b]], rows_v[b],
                                  sem_g[b]).wait()

        def start_wb(c, b):
            pltpu.async_copy(rows_v[b], out_hbm.at[pl.ds(c * B + boff, CH)],
                             sem_w[b])

        def wait_wb(b):
            pltpu.make_async_copy(rows_v[b], out_hbm.at[pl.ds(boff, CH)],
                                  sem_w[b]).wait()

        def step(r, b, do_input, do_waitwb):
            ib = (b + GD) % NBUF
            if do_input:
                if do_waitwb:
                    wait_wb(ib)
                start_gather(r + GD, ib)
            wait_gather(b)
            start_wb(r, b)

        for c in range(GD):
            start_gather(c, c)
        for b in range(NBUF):
            step(b, b, True, b >= WD)

        def outer(o, carry):
            r0 = o * NBUF
            for b in range(NBUF):
                step(r0 + b, b, True, True)
            return carry

        lax.fori_loop(1, n_outer - 1, outer, 0)

        r0 = (n_outer - 1) * NBUF
        for b in range(NBUF):
            step(r0 + b, b, b < WD, True)
        for b in range(NBUF):
            wait_wb(b)

    return gather_kernel


def kernel(weight, nodes):
    V, D = weight.shape
    Bt, J = nodes.shape
    idx = jnp.reshape(jnp.transpose(nodes), (J * Bt,))
    table_v = (weight.reshape(V // 32, 4, 8, D)
               .transpose(0, 2, 1, 3)
               .reshape(V, D))
    out2d = _gather_call(V, D, J, Bt, NBUF=5, GD=2)(table_v, idx)
    out_t = out2d.reshape(J, Bt, D)
    return jnp.transpose(out_t, (1, 0, 2))


# speedup vs baseline: 1.0412x; 1.0412x over previous
import functools

import jax
import jax.numpy as jnp
from jax import lax
from jax.experimental import pallas as pl
from jax.experimental.pallas import tpu as pltpu
from jax.experimental.pallas import tpu_sc as plsc

NC = 2
NS = 16
NW = NC * NS
L = 16

_MESH = dict(core_axis_name="c", subcore_axis_name="s", num_cores=NC)


def _table_call(V, D):
    CW = 128
    n_units = V // CW
    TAIL = V - n_units * CW
    nk = (n_units + NW - 1) // NW

    @functools.partial(
        pl.kernel,
        out_type=jax.ShapeDtypeStruct((V * D,), jnp.float32),
        mesh=plsc.VectorSubcoreMesh(**_MESH),
        scratch_types=[
            pltpu.VMEM((D, CW), jnp.float32),
            pltpu.VMEM((D, CW), jnp.float32),
            pltpu.VMEM((CW * D,), jnp.float32),
            pltpu.VMEM((CW * D,), jnp.float32),
            pltpu.VMEM((D, TAIL), jnp.float32),
            pltpu.SemaphoreType.DMA,
            pltpu.SemaphoreType.DMA,
            pltpu.SemaphoreType.DMA,
            pltpu.SemaphoreType.DMA,
        ],
        compiler_params=pltpu.CompilerParams(use_tc_tiling_on_sc=True,
                                             needs_layout_passes=False),
    )
    def table_kernel(wt_hbm, tail_hbm, out_hbm, src0, src1, dst0, dst1,
                     tail_v, si0, si1, so0, so1):
        wid = lax.axis_index("s") * NC + lax.axis_index("c")
        srcs, dsts = (src0, src1), (dst0, dst1)
        sis, sos = (si0, si1), (so0, so1)
        iota32 = lax.iota(jnp.int32, L) * D

        def col0(k):
            return (wid + k * NW) * CW

        def start_read(k, p):
            pltpu.async_copy(wt_hbm.at[:, pl.ds(col0(k), CW)], srcs[p],
                             sis[p])

        def transpose(p, cw=CW):
            pltpu.make_async_copy(wt_hbm.at[:, pl.ds(0, CW)], srcs[p],
                                  sis[p]).wait()
            for c0 in range(0, cw, L):
                for d in range(D):
                    x = srcs[p][d, pl.ds(c0, L)]
                    plsc.store_scatter(dsts[p], [iota32 + (c0 * D + d)], x)

        def start_write(k, p):
            pltpu.async_copy(dsts[p], out_hbm.at[pl.ds(col0(k) * D, CW * D)],
                             sos[p])

        def wait_write(p):
            pltpu.make_async_copy(dsts[p], out_hbm.at[pl.ds(0, CW * D)],
                                  sos[p]).wait()

        @pl.when(wid < n_units)
        def _prime():
            start_read(0, 0)

        def outer(k2, carry):
            for par in range(2):
                k = k2 * 2 + par

                @pl.when(wid + (k + 1) * NW < n_units)
                def _next():
                    start_read(k + 1, 1 - par)

                @pl.when(wid + k * NW < n_units)
                def _work():
                    @pl.when(k2 >= 1)
                    def _():
                        wait_write(par)
                    transpose(par)
                    start_write(k, par)

            return carry

        nk2 = (nk + 1) // 2
        lax.fori_loop(0, nk2, outer, 0)
        for par in range(2):
            @pl.when(wid + (nk2 * 2 - 2 + par) * NW < n_units)
            def _():
                wait_write(par)

        if TAIL:
            @pl.when(wid == n_units % NW)
            def _tail():
                pltpu.sync_copy(tail_hbm, tail_v)
                for c0 in range(0, TAIL, L):
                    for d in range(D):
                        x = tail_v[d, pl.ds(c0, L)]
                        plsc.store_scatter(dst0, [iota32 + (c0 * D + d)], x)
                pltpu.sync_copy(dst0.at[pl.ds(0, TAIL * D)],
                                out_hbm.at[pl.ds(n_units * CW * D, TAIL * D)])

    return table_kernel


def _gather_call(V, D, J, B, NBUF, GD):
    CH = B // NW
    n_ch = J
    WD = NBUF - GD
    assert 1 <= GD < NBUF and n_ch % NBUF == 0
    n_outer = n_ch // NBUF
    assert n_outer >= 2

    scratch = (
        [pltpu.VMEM((CH,), jnp.int32) for _ in range(NBUF)]
        + [pltpu.VMEM((CH, D), jnp.float32) for _ in range(NBUF)]
        + [pltpu.SemaphoreType.DMA for _ in range(2 * NBUF)]
    )

    @functools.partial(
        pl.kernel,
        out_type=jax.ShapeDtypeStruct((J * B, D), jnp.float32),
        mesh=plsc.VectorSubcoreMesh(**_MESH),
        scratch_types=scratch,
        compiler_params=pltpu.CompilerParams(use_tc_tiling_on_sc=False),
    )
    def gather_kernel(table_hbm, idx_hbm, out_hbm, *refs):
        idx_v = refs[0:NBUF]
        rows_v = refs[NBUF:2 * NBUF]
        sem_g = refs[2 * NBUF:3 * NBUF]
        sem_w = refs[3 * NBUF:4 * NBUF]
        wid = lax.axis_index("s") * NC + lax.axis_index("c")
        boff = wid * CH

        def start_gather(c, b):
            pltpu.sync_copy(idx_hbm.at[pl.ds(c * B + boff, CH)], idx_v[b])
            pltpu.async_copy(table_hbm.at[idx_v[b]], rows_v[b], sem_g[b])

        def wait_gather(b):
            pltpu.make_async_copy(table_hbm.at[idx_v[b]], rows_v[b],
                                  sem_g[b]).wait()

        def start_wb(c, b):
            pltpu.async_copy(rows_v[b], out_hbm.at[pl.ds(c * B + boff, CH)],
                             sem_w[b])

        def wait_wb(b):
            pltpu.make_async_copy(rows_v[b], out_hbm.at[pl.ds(boff, CH)],
                                  sem_w[b]).wait()

        def step(r, b, do_input, do_waitwb):
            ib = (b + GD) % NBUF
            if do_input:
                if do_waitwb:
                    wait_wb(ib)
                start_gather(r + GD, ib)
            wait_gather(b)
            start_wb(r, b)

        for c in range(GD):
            start_gather(c, c)
        for b in range(NBUF):
            step(b, b, True, b >= WD)

        def outer(o, carry):
            r0 = o * NBUF
            for b in range(NBUF):
                step(r0 + b, b, True, True)
            return carry

        lax.fori_loop(1, n_outer - 1, outer, 0)

        r0 = (n_outer - 1) * NBUF
        for b in range(NBUF):
            step(r0 + b, b, b < WD, True)
        for b in range(NBUF):
            wait_wb(b)

    return gather_kernel


def kernel(weight, nodes):
    V, D = weight.shape
    Bt, J = nodes.shape
    idx = jnp.reshape(jnp.transpose(nodes), (J * Bt,))
    w_t = jnp.transpose(weight)
    n_main = (V // 128) * 128
    table_lin = _table_call(V, D)(w_t, w_t[:, n_main:]).reshape(V, D)
    out2d = _gather_call(V, D, J, Bt, NBUF=5, GD=2)(table_lin, idx)
    out_t = out2d.reshape(J, Bt, D)
    return jnp.transpose(out_t, (1, 0, 2))


# speedup vs baseline: 1.1821x; 1.1354x over previous
import functools

import jax
import jax.numpy as jnp
from jax import lax
from jax.experimental import pallas as pl
from jax.experimental.pallas import tpu as pltpu
from jax.experimental.pallas import tpu_sc as plsc

NC = 2
NS = 16
NW = NC * NS
L = 16

_MESH = dict(core_axis_name="c", subcore_axis_name="s", num_cores=NC)


def _table_call(V, D):
    CW = 128
    n_units = V // CW
    TAIL = V - n_units * CW
    nk = (n_units + NW - 1) // NW

    @functools.partial(
        pl.kernel,
        out_type=jax.ShapeDtypeStruct((V * D,), jnp.float32),
        mesh=plsc.VectorSubcoreMesh(**_MESH),
        scratch_types=[
            pltpu.VMEM((D, CW), jnp.float32),
            pltpu.VMEM((D, CW), jnp.float32),
            pltpu.VMEM((CW * D,), jnp.float32),
            pltpu.VMEM((CW * D,), jnp.float32),
            pltpu.VMEM((D, TAIL), jnp.float32),
            pltpu.SemaphoreType.DMA,
            pltpu.SemaphoreType.DMA,
            pltpu.SemaphoreType.DMA,
            pltpu.SemaphoreType.DMA,
        ],
        compiler_params=pltpu.CompilerParams(use_tc_tiling_on_sc=True,
                                             needs_layout_passes=False),
    )
    def table_kernel(wt_hbm, tail_hbm, out_hbm, src0, src1, dst0, dst1,
                     tail_v, si0, si1, so0, so1):
        wid = lax.axis_index("s") * NC + lax.axis_index("c")
        srcs, dsts = (src0, src1), (dst0, dst1)
        sis, sos = (si0, si1), (so0, so1)
        iota32 = lax.iota(jnp.int32, L) * D

        def col0(k):
            return (wid + k * NW) * CW

        def start_read(k, p):
            pltpu.async_copy(wt_hbm.at[:, pl.ds(col0(k), CW)], srcs[p],
                             sis[p])

        def transpose(p, cw=CW):
            pltpu.make_async_copy(wt_hbm.at[:, pl.ds(0, CW)], srcs[p],
                                  sis[p]).wait()
            for c0 in range(0, cw, L):
                for d in range(D):
                    x = srcs[p][d, pl.ds(c0, L)]
                    plsc.store_scatter(dsts[p], [iota32 + (c0 * D + d)], x)

        def start_write(k, p):
            pltpu.async_copy(dsts[p], out_hbm.at[pl.ds(col0(k) * D, CW * D)],
                             sos[p])

        def wait_write(p):
            pltpu.make_async_copy(dsts[p], out_hbm.at[pl.ds(0, CW * D)],
                                  sos[p]).wait()

        @pl.when(wid < n_units)
        def _prime():
            start_read(0, 0)

        def outer(k2, carry):
            for par in range(2):
                k = k2 * 2 + par

                @pl.when(wid + (k + 1) * NW < n_units)
                def _next():
                    start_read(k + 1, 1 - par)

                @pl.when(wid + k * NW < n_units)
                def _work():
                    @pl.when(k2 >= 1)
                    def _():
                        wait_write(par)
                    transpose(par)
                    start_write(k, par)

            return carry

        nk2 = (nk + 1) // 2
        lax.fori_loop(0, nk2, outer, 0)
        for par in range(2):
            @pl.when(wid + (nk2 * 2 - 2 + par) * NW < n_units)
            def _():
                wait_write(par)

        if TAIL:
            @pl.when(wid == n_units % NW)
            def _tail():
                pltpu.sync_copy(tail_hbm, tail_v)
                for c0 in range(0, TAIL, L):
                    for d in range(D):
                        x = tail_v[d, pl.ds(c0, L)]
                        plsc.store_scatter(dst0, [iota32 + (c0 * D + d)], x)
                pltpu.sync_copy(dst0.at[pl.ds(0, TAIL * D)],
                                out_hbm.at[pl.ds(n_units * CW * D, TAIL * D)])

    return table_kernel


def _gather_call(V, D, J, B, NBUF, GD):
    CH = B // NW
    n_ch = J
    WD = NBUF - GD
    assert 1 <= GD < NBUF and n_ch % NBUF == 0
    n_outer = n_ch // NBUF
    assert n_outer >= 2

    scratch = (
        [pltpu.VMEM((CH,), jnp.int32) for _ in range(NBUF)]
        + [pltpu.VMEM((CH, D), jnp.float32) for _ in range(NBUF)]
        + [pltpu.SemaphoreType.DMA for _ in range(2 * NBUF)]
    )

    @functools.partial(
        pl.kernel,
        out_type=jax.ShapeDtypeStruct((J * B, D), jnp.float32),
        mesh=plsc.VectorSubcoreMesh(**_MESH),
        scratch_types=scratch,
        compiler_params=pltpu.CompilerParams(use_tc_tiling_on_sc=False),
    )
    def gather_kernel(table_hbm, idx_hbm, out_hbm, *refs):
        idx_v = refs[0:NBUF]
        rows_v = refs[NBUF:2 * NBUF]
        sem_g = refs[2 * NBUF:3 * NBUF]
        sem_w = refs[3 * NBUF:4 * NBUF]
        wid = lax.axis_index("s") * NC + lax.axis_index("c")
        boff = wid * CH

        def start_gather(c, b):
            pltpu.sync_copy(idx_hbm.at[pl.ds(c * B + boff, CH)], idx_v[b])
            pltpu.async_copy(table_hbm.at[idx_v[b]], rows_v[b], sem_g[b])

        def wait_gather(b):
            pltpu.make_async_copy(table_hbm.at[idx_v[b]], rows_v[b],
                                  sem_g[b]).wait()

        def start_wb(c, b):
            pltpu.async_copy(rows_v[b], out_hbm.at[pl.ds(c * B + boff, CH)],
                             sem_w[b])

        def wait_wb(b):
            pltpu.make_async_copy(rows_v[b], out_hbm.at[pl.ds(boff, CH)],
                                  sem_w[b]).wait()

        def step(r, b, do_input, do_waitwb):
            ib = (b + GD) % NBUF
            if do_input:
                if do_waitwb:
                    wait_wb(ib)
                start_gather(r + GD, ib)
            wait_gather(b)
            start_wb(r, b)

        for c in range(GD):
            start_gather(c, c)
        for b in range(NBUF):
            step(b, b, True, b >= WD)

        def outer(o, carry):
            r0 = o * NBUF
            for b in range(NBUF):
                step(r0 + b, b, True, True)
            return carry

        lax.fori_loop(1, n_outer - 1, outer, 0)

        r0 = (n_outer - 1) * NBUF
        for b in range(NBUF):
            step(r0 + b, b, b < WD, True)
        for b in range(NBUF):
            wait_wb(b)

    return gather_kernel


def kernel(weight, nodes):
    V, D = weight.shape
    Bt, J = nodes.shape
    idx = jnp.reshape(jnp.transpose(nodes), (J * Bt,))
    out2d = _gather_call(V, D, J, Bt, NBUF=5, GD=2)(weight, idx)
    out_t = out2d.reshape(J, Bt, D)
    return jnp.transpose(out_t, (1, 0, 2))
